# baseline (speedup 1.0000x reference)
import functools

import jax
import jax.numpy as jnp
from jax import lax
from jax.experimental import pallas as pl
from jax.experimental.pallas import tpu as pltpu
from jax.experimental.pallas import tpu_sc as plsc

_SEQ = 1024
_D = 64
_NP = 128
_NK = 16
_PH_PER_W = _NP // 32
_ROWS_PER_PH = _SEQ // _NP
_WIN = _NP * _NK


def _tc_phase_windows(fut_pad):
    def body(fut_ref, out_ref):
        p = pl.program_id(0)
        rolled = pltpu.roll(fut_ref[:], lax.rem(_WIN + _NP - p, _WIN + _NP), 1)
        for dt in range(8):
            for kk in range(_NK):
                out_ref[0, dt, kk] = rolled[8 * dt:8 * dt + 8,
                                            _NP * kk:_NP * (kk + 1)]

    return pl.pallas_call(
        body,
        grid=(_NP,),
        in_specs=[pl.BlockSpec((_D, _WIN + _NP), lambda p: (0, 0))],
        out_specs=pl.BlockSpec((1, 8, _NK, 8, _NP), lambda p: (p, 0, 0, 0, 0)),
        out_shape=jax.ShapeDtypeStruct((_NP, 8, _NK, 8, _NP), jnp.float32),
    )(fut_pad)


def _sc_toeplitz(s_tbl):
    mesh = plsc.VectorSubcoreMesh(core_axis_name="c", subcore_axis_name="s",
                                  num_cores=2)

    @functools.partial(
        pl.kernel,
        mesh=mesh,
        out_type=jax.ShapeDtypeStruct((_SEQ, 8, 8, 8, _NP), jnp.float32),
        scratch_types=[
            pltpu.VMEM((_NK, 8, _NP), jnp.float32),
            pltpu.SemaphoreType.DMA,
        ],
    )
    def k(s_hbm, out_hbm, slab, sem):
        c = lax.axis_index("c")
        s = lax.axis_index("s")
        wid = c * 16 + s

        def group_body(g, carry):
            p = wid * _PH_PER_W + lax.div(g, 8)
            dt = lax.rem(g, 8)
            pltpu.sync_copy(s_hbm.at[p, dt], slab)

            def m_body(m, carry):
                i = _SEQ - 1 - _NP * m - p
                pltpu.async_copy(
                    slab.at[pl.ds(m, 8)], out_hbm.at[i, dt], sem
                ).wait()
                return carry

            return lax.fori_loop(0, _ROWS_PER_PH, m_body, carry)

        lax.fori_loop(0, _PH_PER_W * 8, group_body, 0)

    return k(s_tbl)


def kernel(seq_len, relative_position_matrix):
    del seq_len
    fu = jnp.flip(relative_position_matrix, axis=0)[_SEQ:3 * _SEQ, :]
    fut_pad = jnp.pad(fu.T, ((0, 0), (0, _NP)))
    s_tbl = _tc_phase_windows(fut_pad)
    out5 = _sc_toeplitz(s_tbl)
    return out5.transpose(0, 2, 4, 1, 3).reshape(_SEQ, _SEQ, _D)


# speedup vs baseline: 1.1696x; 1.1696x over previous
import functools

import jax
import jax.numpy as jnp
from jax import lax
from jax.experimental import pallas as pl
from jax.experimental.pallas import tpu as pltpu
from jax.experimental.pallas import tpu_sc as plsc

_SEQ = 1024
_D = 64
_NP = 128
_NK = 15
_PH_PER_W = _NP // 32
_ROWS_PER_PH = _SEQ // _NP
_WIN = _NP * _NK


def _tc_phase_windows(fut_pad):
    def body(fut_ref, out_ref):
        p = pl.program_id(0)
        rolled = pltpu.roll(fut_ref[:], lax.rem(_WIN + _NP - p, _WIN + _NP), 1)
        for dt in range(8):
            for kk in range(_NK):
                out_ref[0, dt, kk] = rolled[8 * dt:8 * dt + 8,
                                            _NP * kk:_NP * (kk + 1)]

    return pl.pallas_call(
        body,
        grid=(_NP,),
        in_specs=[pl.BlockSpec((_D, _WIN + _NP), lambda p: (0, 0))],
        out_specs=pl.BlockSpec((1, 8, _NK, 8, _NP), lambda p: (p, 0, 0, 0, 0)),
        out_shape=jax.ShapeDtypeStruct((_NP, 8, _NK, 8, _NP), jnp.float32),
    )(fut_pad)


def _sc_toeplitz(s_tbl):
    mesh = plsc.VectorSubcoreMesh(core_axis_name="c", subcore_axis_name="s",
                                  num_cores=2)

    @functools.partial(
        pl.kernel,
        mesh=mesh,
        out_type=jax.ShapeDtypeStruct((_SEQ, 8, 8, 8, _NP), jnp.float32),
        scratch_types=[
            pltpu.VMEM((2, 4, _NK, 8, _NP), jnp.float32),
            pltpu.SemaphoreType.DMA,
            pltpu.SemaphoreType.DMA,
        ],
    )
    def k(s_hbm, out_hbm, slab, sem, sem_l):
        c = lax.axis_index("c")
        s = lax.axis_index("s")
        wid = c * 16 + s
        n_grp = _PH_PER_W * 2

        def load_args(g, buf):
            p = wid * _PH_PER_W + lax.div(g, 2)
            h = lax.rem(g, 2)
            return s_hbm.at[p, pl.ds(4 * h, 4)], slab.at[buf]

        src0, dst0 = load_args(0, 0)
        pltpu.async_copy(src0, dst0, sem_l)

        def group_body(g, carry):
            p = wid * _PH_PER_W + lax.div(g, 2)
            h = lax.rem(g, 2)
            buf = lax.rem(g, 2)
            src, dst = load_args(g, buf)
            pltpu.make_async_copy(src, dst, sem_l).wait()

            @pl.when(g < n_grp - 1)
            def _():
                src2, dst2 = load_args(g + 1, 1 - buf)
                pltpu.async_copy(src2, dst2, sem_l)

            cps = []
            for m in range(_ROWS_PER_PH):
                i = _SEQ - 1 - _NP * m - p
                cps.append(
                    pltpu.async_copy(
                        slab.at[buf, :, pl.ds(m, 8)],
                        out_hbm.at[i, pl.ds(4 * h, 4)],
                        sem,
                    )
                )
            for cp in cps:
                cp.wait()
            return carry

        lax.fori_loop(0, n_grp, group_body, 0)

    return k(s_tbl)


def kernel(seq_len, relative_position_matrix):
    del seq_len
    fu = jnp.flip(relative_position_matrix, axis=0)[_SEQ:3 * _SEQ, :]
    fut_pad = jnp.pad(fu.T, ((0, 0), (0, _NP)))
    s_tbl = _tc_phase_windows(fut_pad)
    out5 = _sc_toeplitz(s_tbl)
    return out5.transpose(0, 2, 4, 1, 3).reshape(_SEQ, _SEQ, _D)
